# baseline (speedup 1.0000x reference)
import functools

import jax
import jax.numpy as jnp
from jax import lax
from jax.experimental import pallas as pl
from jax.experimental.pallas import tpu as pltpu
from jax.experimental.pallas import tpu_sc as plsc

B = 16384
K = 32
ROWS_PER_SLAB = 4
NC = 2
NS = 16
NW = NC * NS
BPW = B // NW
CHUNK = 128
NCHUNK = BPW // CHUNK
GROUPS = CHUNK // 16

_mesh = plsc.VectorSubcoreMesh(core_axis_name="c", subcore_axis_name="s")


@functools.partial(
    pl.kernel,
    mesh=_mesh,
    compiler_params=pltpu.CompilerParams(needs_layout_passes=False),
    out_type=jax.ShapeDtypeStruct((B,), jnp.float32),
    scratch_types=[
        pltpu.VMEM((NCHUNK, CHUNK), jnp.int32),
        pltpu.VMEM((NCHUNK, CHUNK), jnp.int32),
        pltpu.VMEM((NCHUNK, CHUNK), jnp.int32),
        pltpu.VMEM((NCHUNK, CHUNK), jnp.int32),
        pltpu.VMEM((CHUNK, 128), jnp.float32),
        pltpu.VMEM((CHUNK, 128), jnp.float32),
        pltpu.VMEM((CHUNK, 128), jnp.float32),
        pltpu.VMEM((CHUNK, 128), jnp.float32),
        pltpu.VMEM((BPW,), jnp.float32),
        pltpu.SemaphoreType.DMA,
        pltpu.SemaphoreType.DMA,
    ],
)
def _mf_kernel(uidx_hbm, iidx_hbm, user_hbm, item_hbm, out_hbm,
               uidx_v, iidx_v, uslab_v, islab_v,
               ub0, ub1, ib0, ib1, out_v, sem0, sem1):
    wid = lax.axis_index("s") * NC + lax.axis_index("c")
    base = wid * BPW

    pltpu.sync_copy(uidx_hbm.at[pl.ds(wid * NCHUNK, NCHUNK)], uidx_v)
    pltpu.sync_copy(iidx_hbm.at[pl.ds(wid * NCHUNK, NCHUNK)], iidx_v)

    for c in range(NCHUNK):
        for g in range(GROUPS):
            s = pl.ds(g * 16, 16)
            uslab_v[c, s] = jax.lax.shift_right_logical(uidx_v[c, s], 2)
            islab_v[c, s] = jax.lax.shift_right_logical(iidx_v[c, s], 2)

    ubufs = (ub0, ub1)
    ibufs = (ib0, ib1)
    sems = (sem0, sem1)

    def fire(c):
        cu = pltpu.async_copy(user_hbm.at[uslab_v.at[c]], ubufs[c % 2],
                              sems[c % 2])
        ci = pltpu.async_copy(item_hbm.at[islab_v.at[c]], ibufs[c % 2],
                              sems[c % 2])
        return cu, ci

    pending = fire(0)
    for c in range(NCHUNK):
        nxt = fire(c + 1) if c + 1 < NCHUNK else None
        pending[0].wait()
        pending[1].wait()
        pu, pi = ubufs[c % 2], ibufs[c % 2]
        for g in range(GROUPS):
            s = pl.ds(g * 16, 16)
            rows = lax.iota(jnp.int32, 16) + g * 16
            ucol = jax.lax.shift_left(uidx_v[c, s] & 3, 5)
            icol = jax.lax.shift_left(iidx_v[c, s] & 3, 5)
            acc0 = jnp.zeros((16,), jnp.float32)
            acc1 = jnp.zeros((16,), jnp.float32)
            for k in range(0, K, 2):
                acc0 = acc0 + (plsc.load_gather(pu, [rows, ucol + k])
                               * plsc.load_gather(pi, [rows, icol + k]))
                acc1 = acc1 + (plsc.load_gather(pu, [rows, ucol + (k + 1)])
                               * plsc.load_gather(pi, [rows, icol + (k + 1)]))
            out_v[pl.ds(c * CHUNK + g * 16, 16)] = acc0 + acc1
        pending = nxt

    pltpu.sync_copy(out_v, out_hbm.at[pl.ds(base, BPW)])


def kernel(x, user_emb, item_emb):
    u2 = user_emb.reshape(1000000 * K // 128, 128)
    i2 = item_emb.reshape(1000000 * K // 128, 128)
    uidx = x[:, 0].reshape(B // CHUNK, CHUNK).astype(jnp.int32)
    iidx = x[:, 1].reshape(B // CHUNK, CHUNK).astype(jnp.int32)
    return _mf_kernel(uidx, iidx, u2, i2)
